# baseline (speedup 1.0000x reference)
import jax
import jax.numpy as jnp
from jax import lax
from jax.experimental import pallas as pl
from jax.experimental.pallas import tpu as pltpu
from jax.experimental.pallas import tpu_sc as plsc

_NQ = 256
_B = 8
_T = 8192
_TW = 2048
_TB = 128
_NCHUNK = _TW // _TB


def _sc_body(idx_hbm, out_hbm, idx_v, buf0, buf1, sem0, sem1):
    ns = 16
    nc = 2
    wid = lax.axis_index("s") * nc + lax.axis_index("c")
    b = wid // 4
    tbase = (wid % 4) * _TW

    pltpu.sync_copy(idx_hbm.at[b, pl.ds(tbase, _TW)], idx_v)

    zeros16 = jnp.zeros((16,), jnp.float32)
    ones16 = jnp.ones((16,), jnp.float32)
    iota16 = lax.iota(jnp.int32, 16)

    def _zbody(i, carry):
        r = i >> 3
        col = (i & 7) * 16
        buf0[r, pl.ds(col, 16)] = zeros16
        buf1[r, pl.ds(col, 16)] = zeros16
        return carry

    lax.fori_loop(0, (_NQ * _TB) // 16, _zbody, 0)

    def _scatter(buf, c, vals):
        for j in range(_TB // 16):
            v_idx = idx_v[pl.ds(c * _TB + j * 16, 16)]
            col = iota16 + (j * 16)
            plsc.store_scatter(buf, [v_idx, col], vals)

    bufs = (buf0, buf1)
    sems = (sem0, sem1)
    copies = [None, None]
    for c in range(_NCHUNK):
        k = c & 1
        buf = bufs[k]
        if c >= 2:
            copies[k].wait()
            _scatter(buf, c - 2, zeros16)
        _scatter(buf, c, ones16)
        cp = pltpu.make_async_copy(
            buf, out_hbm.at[b, :, pl.ds(tbase + c * _TB, _TB)], sems[k]
        )
        cp.start()
        copies[k] = cp
    copies[0].wait()
    copies[1].wait()


def kernel(in_snd_slice, quant_onehot):
    idx = in_snd_slice.astype(jnp.int32)
    mesh = plsc.VectorSubcoreMesh(core_axis_name="c", subcore_axis_name="s")
    k = pl.kernel(
        _sc_body,
        mesh=mesh,
        out_type=jax.ShapeDtypeStruct((_B, _NQ, _T), jnp.float32),
        scratch_types=[
            pltpu.VMEM((_TW,), jnp.int32),
            pltpu.VMEM((_NQ, _TB), jnp.float32),
            pltpu.VMEM((_NQ, _TB), jnp.float32),
            pltpu.SemaphoreType.DMA,
            pltpu.SemaphoreType.DMA,
        ],
        compiler_params=pltpu.CompilerParams(needs_layout_passes=False),
    )
    return k(idx)
